# baseline (speedup 1.0000x reference)
import functools

import jax
import jax.numpy as jnp
from jax import lax
from jax.experimental import pallas as pl
from jax.experimental.pallas import tpu as pltpu
from jax.experimental.pallas import tpu_sc as plsc

B = 16384
F = 26
V = 100000
D = 32
ND = 16

NC = 2
NS = 16
NW = NC * NS
NSUP = F * V * D // 128
HV = V // 2
PROWS = 400
BCH = 2048
RPT = B // NW
CH = 16
IPC = CH * F
SUB = 104
NSUB = IPC // SUB


def _k1_body(idxT, e2w, sv_out, sq_out, pbuf, icol, svb, sqb):
    core = lax.axis_index("c")
    sub = lax.axis_index("s")
    d = sub * NC + core
    lanes = lax.iota(jnp.int32, 16)
    zero = jnp.zeros((16,), jnp.float32)

    def zboth(i, _):
        sl = pl.ds(i * 16, 16)
        svb[sl] = zero
        sqb[sl] = zero
        return 0

    lax.fori_loop(0, B // 16, zboth, 0)

    def field_body(f, _):
        p = f * D + d
        elem0 = p * V

        def half_body(h, _):
            eh = elem0 + h * HV
            row0 = eh // 128
            row0b = jnp.minimum((row0 // 8) * 8, NSUP - PROWS)
            lead = eh - row0b * 128
            pltpu.sync_copy(e2w.at[pl.ds(row0b, PROWS)], pbuf)
            lo = h * HV

            def chunk_body(ch, _):
                b0 = ch * BCH
                pltpu.sync_copy(idxT.at[f, pl.ds(b0, BCH)], icol)

                def g_body(g, _):
                    iv = icol[pl.ds(g * 16, 16)]
                    rel = iv - lo
                    m = (rel >= 0) & (rel < HV)
                    pos = jnp.clip(rel, 0, HV - 1) + lead
                    x = plsc.load_gather(pbuf, [pos // 128, pos % 128])
                    x = jnp.where(m, x, 0.0)
                    sl = pl.ds(b0 + g * 16, 16)
                    svb[sl] = svb[sl] + x
                    sqb[sl] = sqb[sl] + x * x
                    return 0

                lax.fori_loop(0, BCH // 16, g_body, 0)
                return 0

            lax.fori_loop(0, B // BCH, chunk_body, 0)
            return 0

        lax.fori_loop(0, 2, half_body, 0)
        return 0

    lax.fori_loop(0, F, field_body, 0)
    pltpu.sync_copy(svb, sv_out.at[d])
    pltpu.sync_copy(sqb, sq_out.at[d])


@functools.partial(
    pl.kernel,
    out_type=(jax.ShapeDtypeStruct((D, B), jnp.float32),
              jax.ShapeDtypeStruct((NW, B), jnp.float32)),
    mesh=plsc.VectorSubcoreMesh(core_axis_name="c", subcore_axis_name="s"),
    compiler_params=pltpu.CompilerParams(needs_layout_passes=False,
                                         use_tc_tiling_on_sc=False),
    scratch_types=[
        pltpu.VMEM((PROWS, 128), jnp.float32),
        pltpu.VMEM((BCH,), jnp.int32),
        pltpu.VMEM((B,), jnp.float32),
        pltpu.VMEM((B,), jnp.float32),
    ],
)
def _fm2_k1(idxT, e2w, sv_out, sq_out, *rest):
    _k1_body(idxT, e2w, sv_out, sq_out, *rest)


def _k2_body(sv_in, sq_in, idx_hbm, e1_hbm, xd_hbm, wd_hbm, bd_hbm, out_hbm,
             svb, sqs, sq5, idxb, e1b, xdb, wdb, bdb, outb, sem_sv, sem_e1):
    core = lax.axis_index("c")
    sub = lax.axis_index("s")
    w = sub * NC + core
    rows0 = w * RPT
    ibase = rows0 * F
    lanes = lax.iota(jnp.int32, 16)
    lanesF = lanes * F
    zero = jnp.zeros((16,), jnp.float32)

    for d in range(D):
        pltpu.async_copy(sv_in.at[d, pl.ds(rows0, RPT)], svb.at[d], sem_sv)
    for q in range(NW):
        pltpu.async_copy(sq_in.at[q, pl.ds(rows0, RPT)], sqs.at[q], sem_sv)
    pltpu.sync_copy(xd_hbm.at[pl.ds(rows0 * ND, RPT * ND)], xdb)
    pltpu.sync_copy(wd_hbm, wdb)
    pltpu.sync_copy(bd_hbm, bdb)
    pltpu.make_async_copy(sv_in.at[pl.ds(0, D), pl.ds(0, RPT)],
                          svb, sem_sv).wait()
    pltpu.make_async_copy(sq_in.at[pl.ds(0, NW), pl.ds(0, RPT)],
                          sqs, sem_sv).wait()

    def sq_body(i, _):
        sl = pl.ds(i * 16, 16)

        def srow(q, acc):
            return acc + sqs[q, sl]

        sq5[sl] = lax.fori_loop(0, NW, srow, zero)
        return 0

    lax.fori_loop(0, RPT // 16, sq_body, 0)

    wv = wdb[...]
    bdv = bdb[...]

    gdn = lax.GatherDimensionNumbers(
        offset_dims=(), collapsed_slice_dims=(0,), start_index_map=(0,))

    def lane_shuffle(x, idx):
        return lax.gather(x, idx[:, None], dimension_numbers=gdn,
                          slice_sizes=(1,),
                          mode=lax.GatherScatterMode.PROMISE_IN_BOUNDS)

    def lane_allsum(x):
        for s in (1, 2, 4, 8):
            x = x + lane_shuffle(x, lanes ^ s)
        return x

    for g in range(RPT // CH):
        pltpu.sync_copy(idx_hbm.at[pl.ds(ibase + g * IPC, IPC)],
                        idxb.at[pl.ds(0, IPC)])
        for j in range(NSUB):
            sl = pl.ds(j * SUB, SUB)
            pltpu.async_copy(e1_hbm.at[idxb.at[sl]], e1b.at[sl], sem_e1)

        def row_body(i, rpack):
            r = g * CH + i
            r16 = lanes * 0 + r
            s0 = plsc.load_gather(svb, [lanes, r16])
            s1 = plsc.load_gather(svb, [lanes + 16, r16])
            xv = xdb[pl.ds(r * ND, 16)]
            rvec = 0.5 * (s0 * s0 + s1 * s1) + xv * wv
            return jnp.where(lanes == i, lane_allsum(rvec), rpack)

        rpack = lax.fori_loop(0, CH, row_body, zero)

        pltpu.make_async_copy(e1_hbm.at[pl.ds(0, IPC)], e1b, sem_e1).wait()

        def e1_body(fi, acc):
            return acc + plsc.load_gather(e1b, [lanesF + fi])

        s1sum = lax.fori_loop(0, F, e1_body, zero)
        v = rpack - 0.5 * sq5[pl.ds(g * CH, 16)] + s1sum + bdv
        outb[pl.ds(g * CH, 16)] = 1.0 / (1.0 + jnp.exp(-v))

    pltpu.sync_copy(outb, out_hbm.at[pl.ds(rows0, RPT)])


@functools.partial(
    pl.kernel,
    out_type=jax.ShapeDtypeStruct((B,), jnp.float32),
    mesh=plsc.VectorSubcoreMesh(core_axis_name="c", subcore_axis_name="s"),
    compiler_params=pltpu.CompilerParams(needs_layout_passes=False,
                                         use_tc_tiling_on_sc=False),
    scratch_types=[
        pltpu.VMEM((D, RPT), jnp.float32),
        pltpu.VMEM((NW, RPT), jnp.float32),
        pltpu.VMEM((RPT,), jnp.float32),
        pltpu.VMEM((IPC + 16,), jnp.int32),
        pltpu.VMEM((IPC,), jnp.float32),
        pltpu.VMEM((RPT * ND,), jnp.float32),
        pltpu.VMEM((16,), jnp.float32),
        pltpu.VMEM((16,), jnp.float32),
        pltpu.VMEM((RPT,), jnp.float32),
        pltpu.SemaphoreType.DMA,
        pltpu.SemaphoreType.DMA,
    ],
)
def _fm2_k2(sv_in, sq_in, idx_hbm, e1_hbm, xd_hbm, wd_hbm, bd_hbm, out_hbm,
            *rest):
    _k2_body(sv_in, sq_in, idx_hbm, e1_hbm, xd_hbm, wd_hbm, bd_hbm, out_hbm,
             *rest)


def kernel(X_sparse, X_dense, emb1, emb2, Wd, bd):
    idxT = X_sparse.T.astype(jnp.int32)
    idx_flat = (X_sparse.astype(jnp.int32)
                + jnp.arange(F, dtype=jnp.int32)[None, :] * V).reshape(-1)
    e2w = emb2.transpose(0, 2, 1).reshape(NSUP, 128)
    e1_flat = emb1.reshape(F * V)
    xd_flat = X_dense.reshape(B * ND)
    wd_flat = Wd.reshape(ND)
    bd16 = jnp.broadcast_to(bd, (16,))
    sv, sq = _fm2_k1(idxT, e2w)
    out = _fm2_k2(sv, sq, idx_flat, e1_flat, xd_flat, wd_flat, bd16)
    return out.reshape(B, 1)


# speedup vs baseline: 2.0779x; 2.0779x over previous
import functools

import jax
import jax.numpy as jnp
from jax import lax
from jax.experimental import pallas as pl
from jax.experimental.pallas import tpu as pltpu
from jax.experimental.pallas import tpu_sc as plsc

B = 16384
F = 26
V = 100000
D = 32
ND = 16

NC = 2
NS = 16
NW = NC * NS
NSUP = F * V * D // 128
HV = V // 2
PROWS = 400
BCH = 2048
RPT = B // NW
CH = 16
IPC = CH * F
SUB = 104
NSUB = IPC // SUB


def _k1_body(idxT, e2w, sv_out, sq_out, pbuf, icol, svb, sqb):
    core = lax.axis_index("c")
    sub = lax.axis_index("s")
    d = sub * NC + core
    lanes = lax.iota(jnp.int32, 16)
    zero = jnp.zeros((16,), jnp.float32)

    def zboth(i, _):
        sl = pl.ds(i * 16, 16)
        svb[sl] = zero
        sqb[sl] = zero
        return 0

    lax.fori_loop(0, B // 16, zboth, 0)

    def field_body(f, _):
        p = f * D + d
        elem0 = p * V
        pltpu.sync_copy(idxT.at[f], icol)

        def half_body(h, _):
            eh = elem0 + h * HV
            row0 = eh // 128
            row0b = jnp.minimum((row0 // 8) * 8, NSUP - PROWS)
            lead = eh - row0b * 128
            pltpu.sync_copy(e2w.at[pl.ds(row0b, PROWS)], pbuf)
            lo = h * HV

            def g_body(g, _):
                iv = icol[pl.ds(g * 16, 16)]
                rel = iv - lo
                m = (rel >= 0) & (rel < HV)
                pos = jnp.clip(rel, 0, HV - 1) + lead
                x = plsc.load_gather(pbuf, [pos // 128, pos % 128])
                x = jnp.where(m, x, 0.0)
                sl = pl.ds(g * 16, 16)
                svb[sl] = svb[sl] + x
                sqb[sl] = sqb[sl] + x * x
                return 0

            lax.fori_loop(0, B // 16, g_body, 0)
            return 0

        lax.fori_loop(0, 2, half_body, 0)
        return 0

    lax.fori_loop(0, F, field_body, 0)
    pltpu.sync_copy(svb, sv_out.at[d])
    pltpu.sync_copy(sqb, sq_out.at[d])


@functools.partial(
    pl.kernel,
    out_type=(jax.ShapeDtypeStruct((D, B), jnp.float32),
              jax.ShapeDtypeStruct((NW, B), jnp.float32)),
    mesh=plsc.VectorSubcoreMesh(core_axis_name="c", subcore_axis_name="s"),
    compiler_params=pltpu.CompilerParams(needs_layout_passes=False,
                                         use_tc_tiling_on_sc=False),
    scratch_types=[
        pltpu.VMEM((PROWS, 128), jnp.float32),
        pltpu.VMEM((B,), jnp.int32),
        pltpu.VMEM((B,), jnp.float32),
        pltpu.VMEM((B,), jnp.float32),
    ],
)
def _fm2_k1(idxT, e2w, sv_out, sq_out, *rest):
    _k1_body(idxT, e2w, sv_out, sq_out, *rest)


def _k2_body(sv_in, sq_in, idx_hbm, e1_hbm, xd_hbm, wd_hbm, bd_hbm, out_hbm,
             svb, sqs, sq5, idxb, e1b, xdb, wdb, bdb, outb, sem_sv, sem_e1):
    core = lax.axis_index("c")
    sub = lax.axis_index("s")
    w = sub * NC + core
    rows0 = w * RPT
    ibase = rows0 * F
    lanes = lax.iota(jnp.int32, 16)
    lanesF = lanes * F
    zero = jnp.zeros((16,), jnp.float32)

    for d in range(D):
        pltpu.async_copy(sv_in.at[d, pl.ds(rows0, RPT)], svb.at[d], sem_sv)
    for q in range(NW):
        pltpu.async_copy(sq_in.at[q, pl.ds(rows0, RPT)], sqs.at[q], sem_sv)
    pltpu.sync_copy(xd_hbm.at[pl.ds(rows0 * ND, RPT * ND)], xdb)
    pltpu.sync_copy(wd_hbm, wdb)
    pltpu.sync_copy(bd_hbm, bdb)
    pltpu.make_async_copy(sv_in.at[pl.ds(0, D), pl.ds(0, RPT)],
                          svb, sem_sv).wait()
    pltpu.make_async_copy(sq_in.at[pl.ds(0, NW), pl.ds(0, RPT)],
                          sqs, sem_sv).wait()

    def sq_body(i, _):
        sl = pl.ds(i * 16, 16)

        def srow(q, acc):
            return acc + sqs[q, sl]

        sq5[sl] = lax.fori_loop(0, NW, srow, zero)
        return 0

    lax.fori_loop(0, RPT // 16, sq_body, 0)

    wv = wdb[...]
    bdv = bdb[...]

    gdn = lax.GatherDimensionNumbers(
        offset_dims=(), collapsed_slice_dims=(0,), start_index_map=(0,))

    def lane_shuffle(x, idx):
        return lax.gather(x, idx[:, None], dimension_numbers=gdn,
                          slice_sizes=(1,),
                          mode=lax.GatherScatterMode.PROMISE_IN_BOUNDS)

    def lane_allsum(x):
        for s in (1, 2, 4, 8):
            x = x + lane_shuffle(x, lanes ^ s)
        return x

    for g in range(RPT // CH):
        pltpu.sync_copy(idx_hbm.at[pl.ds(ibase + g * IPC, IPC)],
                        idxb.at[pl.ds(0, IPC)])
        for j in range(NSUB):
            sl = pl.ds(j * SUB, SUB)
            pltpu.async_copy(e1_hbm.at[idxb.at[sl]], e1b.at[sl], sem_e1)

        def row_body(i, rpack):
            r = g * CH + i
            r16 = lanes * 0 + r
            s0 = plsc.load_gather(svb, [lanes, r16])
            s1 = plsc.load_gather(svb, [lanes + 16, r16])
            xv = xdb[pl.ds(r * ND, 16)]
            rvec = 0.5 * (s0 * s0 + s1 * s1) + xv * wv
            return jnp.where(lanes == i, lane_allsum(rvec), rpack)

        rpack = lax.fori_loop(0, CH, row_body, zero)

        pltpu.make_async_copy(e1_hbm.at[pl.ds(0, IPC)], e1b, sem_e1).wait()

        def e1_body(fi, acc):
            return acc + plsc.load_gather(e1b, [lanesF + fi])

        s1sum = lax.fori_loop(0, F, e1_body, zero)
        v = rpack - 0.5 * sq5[pl.ds(g * CH, 16)] + s1sum + bdv
        outb[pl.ds(g * CH, 16)] = 1.0 / (1.0 + jnp.exp(-v))

    pltpu.sync_copy(outb, out_hbm.at[pl.ds(rows0, RPT)])


@functools.partial(
    pl.kernel,
    out_type=jax.ShapeDtypeStruct((B,), jnp.float32),
    mesh=plsc.VectorSubcoreMesh(core_axis_name="c", subcore_axis_name="s"),
    compiler_params=pltpu.CompilerParams(needs_layout_passes=False,
                                         use_tc_tiling_on_sc=False),
    scratch_types=[
        pltpu.VMEM((D, RPT), jnp.float32),
        pltpu.VMEM((NW, RPT), jnp.float32),
        pltpu.VMEM((RPT,), jnp.float32),
        pltpu.VMEM((IPC + 16,), jnp.int32),
        pltpu.VMEM((IPC,), jnp.float32),
        pltpu.VMEM((RPT * ND,), jnp.float32),
        pltpu.VMEM((16,), jnp.float32),
        pltpu.VMEM((16,), jnp.float32),
        pltpu.VMEM((RPT,), jnp.float32),
        pltpu.SemaphoreType.DMA,
        pltpu.SemaphoreType.DMA,
    ],
)
def _fm2_k2(sv_in, sq_in, idx_hbm, e1_hbm, xd_hbm, wd_hbm, bd_hbm, out_hbm,
            *rest):
    _k2_body(sv_in, sq_in, idx_hbm, e1_hbm, xd_hbm, wd_hbm, bd_hbm, out_hbm,
             *rest)


def kernel(X_sparse, X_dense, emb1, emb2, Wd, bd):
    idxT = X_sparse.T.astype(jnp.int32)
    idx_flat = (X_sparse.astype(jnp.int32)
                + jnp.arange(F, dtype=jnp.int32)[None, :] * V).reshape(-1)
    e2w = emb2.transpose(0, 2, 1).reshape(NSUP, 128)
    e1_flat = emb1.reshape(F * V)
    xd_flat = X_dense.reshape(B * ND)
    wd_flat = Wd.reshape(ND)
    bd16 = jnp.broadcast_to(bd, (16,))
    sv, sq = _fm2_k1(idxT, e2w)
    out = _fm2_k2(sv, sq, idx_flat, e1_flat, xd_flat, wd_flat, bd16)
    return out.reshape(B, 1)


# speedup vs baseline: 2.2939x; 1.1039x over previous
import functools

import jax
import jax.numpy as jnp
from jax import lax
from jax.experimental import pallas as pl
from jax.experimental.pallas import tpu as pltpu
from jax.experimental.pallas import tpu_sc as plsc

B = 16384
F = 26
V = 100000
D = 32
ND = 16

NC = 2
NS = 16
NW = NC * NS
NSUP = F * V * D // 128
PROWS = 792
BCH = 2048
RPT = B // NW
CH = 16
IPC = CH * F
SUB = 104
NSUB = IPC // SUB


def _k1_body(idxT, e2w, sv_out, sq_out, pbuf, icol, svb, sqb):
    core = lax.axis_index("c")
    sub = lax.axis_index("s")
    d = sub * NC + core
    lanes = lax.iota(jnp.int32, 16)
    zero = jnp.zeros((16,), jnp.float32)

    def bh_body(bh, _):
        b0 = bh * (B // 2)

        def zboth(i, _):
            sl = pl.ds(i * 16, 16)
            svb[sl] = zero
            sqb[sl] = zero
            return 0

        lax.fori_loop(0, B // 32, zboth, 0)

        def field_body(f, _):
            elem0 = (f * D + d) * V
            row0 = elem0 // 128
            row0b = jnp.minimum((row0 // 8) * 8, NSUP - PROWS)
            lead = elem0 - row0b * 128
            pltpu.sync_copy(idxT.at[f, pl.ds(b0, B // 2)], icol)
            pltpu.sync_copy(e2w.at[pl.ds(row0b, PROWS)], pbuf)

            def g_body(g, _):
                iv = icol[pl.ds(g * 16, 16)]
                pos = iv + lead
                x = plsc.load_gather(pbuf, [pos // 128, pos % 128])
                sl = pl.ds(g * 16, 16)
                svb[sl] = svb[sl] + x
                sqb[sl] = sqb[sl] + x * x
                return 0

            lax.fori_loop(0, B // 32, g_body, 0)
            return 0

        lax.fori_loop(0, F, field_body, 0)
        pltpu.sync_copy(svb, sv_out.at[d, pl.ds(b0, B // 2)])
        pltpu.sync_copy(sqb, sq_out.at[d, pl.ds(b0, B // 2)])
        return 0

    lax.fori_loop(0, 2, bh_body, 0)


@functools.partial(
    pl.kernel,
    out_type=(jax.ShapeDtypeStruct((D, B), jnp.float32),
              jax.ShapeDtypeStruct((NW, B), jnp.float32)),
    mesh=plsc.VectorSubcoreMesh(core_axis_name="c", subcore_axis_name="s"),
    compiler_params=pltpu.CompilerParams(needs_layout_passes=False,
                                         use_tc_tiling_on_sc=False),
    scratch_types=[
        pltpu.VMEM((PROWS, 128), jnp.float32),
        pltpu.VMEM((B // 2,), jnp.int32),
        pltpu.VMEM((B // 2,), jnp.float32),
        pltpu.VMEM((B // 2,), jnp.float32),
    ],
)
def _fm2_k1(idxT, e2w, sv_out, sq_out, *rest):
    _k1_body(idxT, e2w, sv_out, sq_out, *rest)


def _k2_body(sv_in, sq_in, idx_hbm, e1_hbm, xd_hbm, wd_hbm, bd_hbm, out_hbm,
             svb, sqs, sq5, idxb, e1b, xdb, wdb, bdb, outb, sem_sv, sem_e1):
    core = lax.axis_index("c")
    sub = lax.axis_index("s")
    w = sub * NC + core
    rows0 = w * RPT
    ibase = rows0 * F
    lanes = lax.iota(jnp.int32, 16)
    lanesF = lanes * F
    zero = jnp.zeros((16,), jnp.float32)

    for d in range(D):
        pltpu.async_copy(sv_in.at[d, pl.ds(rows0, RPT)], svb.at[d], sem_sv)
    for q in range(NW):
        pltpu.async_copy(sq_in.at[q, pl.ds(rows0, RPT)], sqs.at[q], sem_sv)
    pltpu.sync_copy(xd_hbm.at[pl.ds(rows0 * ND, RPT * ND)], xdb)
    pltpu.sync_copy(wd_hbm, wdb)
    pltpu.sync_copy(bd_hbm, bdb)
    pltpu.make_async_copy(sv_in.at[pl.ds(0, D), pl.ds(0, RPT)],
                          svb, sem_sv).wait()
    pltpu.make_async_copy(sq_in.at[pl.ds(0, NW), pl.ds(0, RPT)],
                          sqs, sem_sv).wait()

    def sq_body(i, _):
        sl = pl.ds(i * 16, 16)

        def srow(q, acc):
            return acc + sqs[q, sl]

        sq5[sl] = lax.fori_loop(0, NW, srow, zero)
        return 0

    lax.fori_loop(0, RPT // 16, sq_body, 0)

    wv = wdb[...]
    bdv = bdb[...]

    gdn = lax.GatherDimensionNumbers(
        offset_dims=(), collapsed_slice_dims=(0,), start_index_map=(0,))

    def lane_shuffle(x, idx):
        return lax.gather(x, idx[:, None], dimension_numbers=gdn,
                          slice_sizes=(1,),
                          mode=lax.GatherScatterMode.PROMISE_IN_BOUNDS)

    def lane_allsum(x):
        for s in (1, 2, 4, 8):
            x = x + lane_shuffle(x, lanes ^ s)
        return x

    for g in range(RPT // CH):
        pltpu.sync_copy(idx_hbm.at[pl.ds(ibase + g * IPC, IPC)],
                        idxb.at[pl.ds(0, IPC)])
        for j in range(NSUB):
            sl = pl.ds(j * SUB, SUB)
            pltpu.async_copy(e1_hbm.at[idxb.at[sl]], e1b.at[sl], sem_e1)

        def row_body(i, rpack):
            r = g * CH + i
            r16 = lanes * 0 + r
            s0 = plsc.load_gather(svb, [lanes, r16])
            s1 = plsc.load_gather(svb, [lanes + 16, r16])
            xv = xdb[pl.ds(r * ND, 16)]
            rvec = 0.5 * (s0 * s0 + s1 * s1) + xv * wv
            return jnp.where(lanes == i, lane_allsum(rvec), rpack)

        rpack = lax.fori_loop(0, CH, row_body, zero)

        pltpu.make_async_copy(e1_hbm.at[pl.ds(0, IPC)], e1b, sem_e1).wait()

        def e1_body(fi, acc):
            return acc + plsc.load_gather(e1b, [lanesF + fi])

        s1sum = lax.fori_loop(0, F, e1_body, zero)
        v = rpack - 0.5 * sq5[pl.ds(g * CH, 16)] + s1sum + bdv
        outb[pl.ds(g * CH, 16)] = 1.0 / (1.0 + jnp.exp(-v))

    pltpu.sync_copy(outb, out_hbm.at[pl.ds(rows0, RPT)])


@functools.partial(
    pl.kernel,
    out_type=jax.ShapeDtypeStruct((B,), jnp.float32),
    mesh=plsc.VectorSubcoreMesh(core_axis_name="c", subcore_axis_name="s"),
    compiler_params=pltpu.CompilerParams(needs_layout_passes=False,
                                         use_tc_tiling_on_sc=False),
    scratch_types=[
        pltpu.VMEM((D, RPT), jnp.float32),
        pltpu.VMEM((NW, RPT), jnp.float32),
        pltpu.VMEM((RPT,), jnp.float32),
        pltpu.VMEM((IPC + 16,), jnp.int32),
        pltpu.VMEM((IPC,), jnp.float32),
        pltpu.VMEM((RPT * ND,), jnp.float32),
        pltpu.VMEM((16,), jnp.float32),
        pltpu.VMEM((16,), jnp.float32),
        pltpu.VMEM((RPT,), jnp.float32),
        pltpu.SemaphoreType.DMA,
        pltpu.SemaphoreType.DMA,
    ],
)
def _fm2_k2(sv_in, sq_in, idx_hbm, e1_hbm, xd_hbm, wd_hbm, bd_hbm, out_hbm,
            *rest):
    _k2_body(sv_in, sq_in, idx_hbm, e1_hbm, xd_hbm, wd_hbm, bd_hbm, out_hbm,
             *rest)


def kernel(X_sparse, X_dense, emb1, emb2, Wd, bd):
    idxT = X_sparse.T.astype(jnp.int32)
    idx_flat = (X_sparse.astype(jnp.int32)
                + jnp.arange(F, dtype=jnp.int32)[None, :] * V).reshape(-1)
    e2w = emb2.transpose(0, 2, 1).reshape(NSUP, 128)
    e1_flat = emb1.reshape(F * V)
    xd_flat = X_dense.reshape(B * ND)
    wd_flat = Wd.reshape(ND)
    bd16 = jnp.broadcast_to(bd, (16,))
    sv, sq = _fm2_k1(idxT, e2w)
    out = _fm2_k2(sv, sq, idx_flat, e1_flat, xd_flat, wd_flat, bd16)
    return out.reshape(B, 1)
